# baseline (speedup 1.0000x reference)
import functools

import jax
import jax.numpy as jnp
from jax import lax
from jax.experimental import pallas as pl
from jax.experimental.pallas import tpu as pltpu
from jax.experimental.pallas import tpu_sc as plsc

B = 16384
C = 1000
NBINS = 10
MINCLAMP = 0.01
MAXCLAMP = 100.0

NC = 2
NS = 16
L = 16
NW = NC * NS
RW = B // NW

BT = 8192
S = B - BT
CW = S // NW
RCH = 200
NCH = C // RCH

CBLK = 2048
NBT = BT // CBLK

EDGES = (0.0, 0.10000000149011612, 0.20000000298023224, 0.30000001192092896,
         0.4000000059604645, 0.5, 0.6000000238418579, 0.699999988079071,
         0.800000011920929, 0.9000000357627869, 1.0)

_MESH = dict(core_axis_name="c", subcore_axis_name="s", num_cores=NC,
             num_subcores=NS)


def _rowmax_tc_body(p_ref, out_ref, mm_ref, z_ref, accn_ref, accx_ref):
    g = pl.program_id(0)
    z_ref[...] = jnp.zeros((C, CBLK), jnp.float32)

    @pl.when(g < NBT)
    def _():
        m = jnp.max(p_ref[...], axis=0).reshape(CBLK // 128, 128)
        out_ref[...] = m

        @pl.when(g == 0)
        def _():
            accn_ref[...] = m
            accx_ref[...] = m

        @pl.when(g > 0)
        def _():
            accn_ref[...] = jnp.minimum(accn_ref[...], m)
            accx_ref[...] = jnp.maximum(accx_ref[...], m)

    @pl.when(g == NBT - 1)
    def _():
        gmn = jnp.min(accn_ref[...])
        gmx = jnp.max(accx_ref[...])
        col = lax.broadcasted_iota(jnp.int32, (8, 128), 1)
        mm_ref[...] = jnp.where(col == 1, gmx, gmn)


def _rowmax_tc(probs_t):
    return pl.pallas_call(
        _rowmax_tc_body,
        grid=(B // CBLK,),
        in_specs=[pl.BlockSpec((C, CBLK),
                               lambda g: (0, jnp.minimum(g, NBT - 1)))],
        out_specs=[
            pl.BlockSpec((CBLK // 128, 128),
                         lambda g: (jnp.minimum(g, NBT - 1), 0)),
            pl.BlockSpec((8, 128), lambda g: (0, 0)),
            pl.BlockSpec((C, CBLK), lambda g: (0, g)),
        ],
        out_shape=[
            jax.ShapeDtypeStruct((BT // 128, 128), jnp.float32),
            jax.ShapeDtypeStruct((8, 128), jnp.float32),
            jax.ShapeDtypeStruct((C, B), jnp.float32),
        ],
        scratch_shapes=[
            pltpu.VMEM((CBLK // 128, 128), jnp.float32),
            pltpu.VMEM((CBLK // 128, 128), jnp.float32),
        ],
    )(probs_t)


@functools.partial(
    pl.kernel,
    out_type=(
        jax.ShapeDtypeStruct((S,), jnp.float32),
        jax.ShapeDtypeStruct((NW, 2, L), jnp.float32),
    ),
    mesh=plsc.VectorSubcoreMesh(**_MESH),
    scratch_types=[
        pltpu.VMEM((RCH, CW), jnp.float32),
        pltpu.VMEM((RCH, CW), jnp.float32),
        pltpu.VMEM((CW,), jnp.float32),
        pltpu.VMEM((2, L), jnp.float32),
        pltpu.SemaphoreType.DMA,
        pltpu.SemaphoreType.DMA,
    ],
    compiler_params=pltpu.CompilerParams(needs_layout_passes=False,
                                         use_tc_tiling_on_sc=True),
)
def _scmax_kernel(pt_hbm, smax_hbm, part_hbm,
                  buf0, buf1, maxes_v, pbuf, sem0, sem1):
    wid = lax.axis_index("c") * NS + lax.axis_index("s")
    c0 = BT + wid * CW

    def start(i, buf, sem):
        pltpu.make_async_copy(
            pt_hbm.at[pl.ds(i * RCH, RCH), pl.ds(c0, CW)], buf, sem).start()

    def wait(buf, sem):
        pltpu.make_async_copy(
            pt_hbm.at[pl.ds(0, RCH), pl.ds(c0, CW)], buf, sem).wait()

    def proc(buf, first):
        def col_body(j, carry):
            a0 = buf[0, pl.ds(j * L, L)]
            a1 = buf[1, pl.ds(j * L, L)]
            a2 = buf[2, pl.ds(j * L, L)]
            a3 = buf[3, pl.ds(j * L, L)]

            def row_body(k, accs):
                b0, b1, b2, b3 = accs
                r = 4 + 4 * k
                return (jnp.maximum(b0, buf[r, pl.ds(j * L, L)]),
                        jnp.maximum(b1, buf[r + 1, pl.ds(j * L, L)]),
                        jnp.maximum(b2, buf[r + 2, pl.ds(j * L, L)]),
                        jnp.maximum(b3, buf[r + 3, pl.ds(j * L, L)]))

            a0, a1, a2, a3 = lax.fori_loop(0, (RCH - 4) // 4, row_body,
                                           (a0, a1, a2, a3))
            a = jnp.maximum(jnp.maximum(a0, a1), jnp.maximum(a2, a3))
            if first:
                maxes_v[pl.ds(j * L, L)] = a
            else:
                maxes_v[pl.ds(j * L, L)] = jnp.maximum(
                    maxes_v[pl.ds(j * L, L)], a)
            return carry

        lax.fori_loop(0, CW // L, col_body, 0)

    start(0, buf0, sem0)
    start(1, buf1, sem1)
    wait(buf0, sem0)
    proc(buf0, True)
    start(2, buf0, sem0)
    wait(buf1, sem1)
    proc(buf1, False)
    start(3, buf1, sem1)
    wait(buf0, sem0)
    proc(buf0, False)
    start(4, buf0, sem0)
    wait(buf1, sem1)
    proc(buf1, False)
    wait(buf0, sem0)
    proc(buf0, False)

    inf = jnp.full((L,), jnp.inf, jnp.float32)

    def red_body(j, carry):
        amin, amax = carry
        m = maxes_v[pl.ds(j * L, L)]
        return jnp.minimum(amin, m), jnp.maximum(amax, m)

    amin, amax = lax.fori_loop(0, CW // L, red_body, (inf, -inf))
    pbuf[0, :] = amin
    pbuf[1, :] = amax
    pltpu.sync_copy(maxes_v, smax_hbm.at[pl.ds(wid * CW, CW)])
    pltpu.sync_copy(pbuf, part_hbm.at[wid])


@functools.partial(
    pl.kernel,
    out_type=jax.ShapeDtypeStruct((B,), jnp.float32),
    mesh=plsc.VectorSubcoreMesh(**_MESH),
    scratch_types=[
        pltpu.VMEM((RW,), jnp.float32),
        pltpu.VMEM((L,), jnp.float32),
        pltpu.VMEM((NW, 2, L), jnp.float32),
        pltpu.VMEM((L,), jnp.float32),
        pltpu.VMEM((L,), jnp.float32),
        pltpu.VMEM((RW,), jnp.float32),
    ],
    compiler_params=pltpu.CompilerParams(needs_layout_passes=False),
)
def _calibrate_sc(tmax_hbm, smax_hbm, mm_hbm, part_hbm, bp_hbm, out_hbm,
                  m_v, mm_v, pr_v, bp_v, sc_v, out_v):
    wid = lax.axis_index("c") * NS + lax.axis_index("s")
    off = jnp.where(wid < NS, wid, wid - NS) * RW

    @pl.when(wid < NS)
    def _():
        pltpu.sync_copy(tmax_hbm.at[pl.ds(off, RW)], m_v)

    @pl.when(wid >= NS)
    def _():
        pltpu.sync_copy(smax_hbm.at[pl.ds(off, RW)], m_v)

    pltpu.sync_copy(mm_hbm.at[0, pl.ds(0, L)], mm_v)
    pltpu.sync_copy(part_hbm, pr_v)
    pltpu.sync_copy(bp_hbm, bp_v.at[pl.ds(0, NBINS)])

    def red_body(w, carry):
        amin, amax = carry
        return (jnp.minimum(amin, pr_v[w, 0, :]),
                jnp.maximum(amax, pr_v[w, 1, :]))

    amin, amax = lax.fori_loop(1, NW, red_body,
                               (pr_v[0, 0, :], pr_v[0, 1, :]))
    mm = mm_v[:]
    gmn = jnp.minimum(jnp.min(amin), mm[0])
    gmx = jnp.maximum(jnp.max(amax), mm[1])
    denom_v = jnp.zeros((L,), jnp.float32) + (gmx - gmn + jnp.float32(1e-12))
    inv = jnp.full((L,), 1.0, jnp.float32) / denom_v

    bp = bp_v[:]
    sc_v[:] = jnp.float32(1.0) / jnp.clip(bp * bp, jnp.float32(MINCLAMP),
                                          jnp.float32(MAXCLAMP))

    def vec_body(k, carry):
        x = (m_v[pl.ds(k * L, L)] - gmn) * inv
        cnt = jnp.zeros((L,), jnp.int32)
        for e in EDGES:
            cnt = cnt + jnp.where(jnp.float32(e) < x, jnp.int32(1),
                                  jnp.int32(0))
        idx = jnp.clip(cnt - 1, 0, NBINS - 1)
        g = plsc.load_gather(sc_v, [idx])
        out_v[pl.ds(k * L, L)] = x * g
        return carry

    lax.fori_loop(0, RW // L, vec_body, 0)
    pltpu.sync_copy(out_v, out_hbm.at[pl.ds(wid * RW, RW)])


def kernel(probabilities, bin_params):
    pt = probabilities.T
    tmax2d, mm2d, zeros_t = _rowmax_tc(pt)
    smax, parts = _scmax_kernel(pt)
    out = _calibrate_sc(tmax2d.reshape(BT), smax, mm2d, parts, bin_params)
    calibrated = zeros_t.T
    return (out, calibrated)


# speedup vs baseline: 1.1301x; 1.1301x over previous
import functools

import jax
import jax.numpy as jnp
from jax import lax
from jax.experimental import pallas as pl
from jax.experimental.pallas import tpu as pltpu
from jax.experimental.pallas import tpu_sc as plsc

B = 16384
C = 1000
NBINS = 10
MINCLAMP = 0.01
MAXCLAMP = 100.0

NC = 2
NS = 16
L = 16
NW = NC * NS
RW = B // NW

EDGES = (0.0, 0.10000000149011612, 0.20000000298023224, 0.30000001192092896,
         0.4000000059604645, 0.5, 0.6000000238418579, 0.699999988079071,
         0.800000011920929, 0.9000000357627869, 1.0)

CBLK = 2048


def _rowmax_tc_body(p_ref, out_ref, mm_ref, z_ref, accn_ref, accx_ref):
    g = pl.program_id(0)
    x = p_ref[...]
    m = jnp.max(x, axis=0).reshape(CBLK // 128, 128)
    out_ref[...] = m
    z_ref[...] = jnp.zeros((C, CBLK), jnp.float32)

    @pl.when(g == 0)
    def _():
        accn_ref[...] = m
        accx_ref[...] = m

    @pl.when(g > 0)
    def _():
        accn_ref[...] = jnp.minimum(accn_ref[...], m)
        accx_ref[...] = jnp.maximum(accx_ref[...], m)

    @pl.when(g == pl.num_programs(0) - 1)
    def _():
        gmn = jnp.min(accn_ref[...])
        gmx = jnp.max(accx_ref[...])
        col = lax.broadcasted_iota(jnp.int32, (8, 128), 1)
        mm_ref[...] = jnp.where(col == 1, gmx, gmn)


def _rowmax_tc(probs_t):
    return pl.pallas_call(
        _rowmax_tc_body,
        grid=(B // CBLK,),
        in_specs=[pl.BlockSpec((C, CBLK), lambda g: (0, g))],
        out_specs=[
            pl.BlockSpec((CBLK // 128, 128), lambda g: (g, 0)),
            pl.BlockSpec((8, 128), lambda g: (0, 0)),
            pl.BlockSpec((C, CBLK), lambda g: (0, g)),
        ],
        out_shape=[
            jax.ShapeDtypeStruct((B // 128, 128), jnp.float32),
            jax.ShapeDtypeStruct((8, 128), jnp.float32),
            jax.ShapeDtypeStruct((C, B), jnp.float32),
        ],
        scratch_shapes=[
            pltpu.VMEM((CBLK // 128, 128), jnp.float32),
            pltpu.VMEM((CBLK // 128, 128), jnp.float32),
        ],
    )(probs_t)


@functools.partial(
    pl.kernel,
    out_type=jax.ShapeDtypeStruct((B,), jnp.float32),
    mesh=plsc.VectorSubcoreMesh(core_axis_name="c", subcore_axis_name="s",
                                num_cores=NC, num_subcores=NS),
    scratch_types=[
        pltpu.VMEM((RW,), jnp.float32),
        pltpu.VMEM((L,), jnp.float32),
        pltpu.VMEM((L,), jnp.float32),
        pltpu.VMEM((L,), jnp.float32),
        pltpu.VMEM((RW,), jnp.float32),
    ],
    compiler_params=pltpu.CompilerParams(needs_layout_passes=False),
)
def _calibrate_sc(maxes_hbm, mm_hbm, bp_hbm, out_hbm,
                  m_v, mm_v, bp_v, sc_v, out_v):
    wid = lax.axis_index("c") * NS + lax.axis_index("s")
    rbase = wid * RW

    pltpu.sync_copy(maxes_hbm.at[pl.ds(rbase, RW)], m_v)
    pltpu.sync_copy(mm_hbm.at[0, pl.ds(0, L)], mm_v)
    pltpu.sync_copy(bp_hbm, bp_v.at[pl.ds(0, NBINS)])

    mm = mm_v[:]
    gmn = mm[0]
    gmx = mm[1]
    denom_v = jnp.zeros((L,), jnp.float32) + (gmx - gmn + jnp.float32(1e-12))
    inv = jnp.full((L,), 1.0, jnp.float32) / denom_v

    bp = bp_v[:]
    sc_v[:] = jnp.float32(1.0) / jnp.clip(bp * bp, jnp.float32(MINCLAMP),
                                          jnp.float32(MAXCLAMP))

    def vec_body(k, carry):
        x = (m_v[pl.ds(k * L, L)] - gmn) * inv
        cnt = jnp.zeros((L,), jnp.int32)
        for e in EDGES:
            cnt = cnt + jnp.where(jnp.float32(e) < x, jnp.int32(1),
                                  jnp.int32(0))
        idx = jnp.clip(cnt - 1, 0, NBINS - 1)
        g = plsc.load_gather(sc_v, [idx])
        out_v[pl.ds(k * L, L)] = x * g
        return carry

    lax.fori_loop(0, RW // L, vec_body, 0)
    pltpu.sync_copy(out_v, out_hbm.at[pl.ds(rbase, RW)])


def kernel(probabilities, bin_params):
    maxes2d, mm2d, zeros_t = _rowmax_tc(probabilities.T)
    out = _calibrate_sc(maxes2d.reshape(B), mm2d, bin_params)
    calibrated = zeros_t.T
    return (out, calibrated)
